# baseline (speedup 1.0000x reference)
import functools

import jax
import jax.numpy as jnp
from jax import lax
from jax.experimental import pallas as pl
from jax.experimental.pallas import tpu as pltpu
from jax.experimental.pallas import tpu_sc as plsc

VOCAB = 1000000
DIM = 64
TERSE_ID = 5
VERBOSE_ID = 7

NC = 2
NS = 16
NW = NC * NS
LANES = 16

GSIZE = 128
GPC = 8
CHUNK = GSIZE * GPC


def _body(idx_hbm, table_hbm, delta_hbm, out_hbm,
          idx_v, rows_v, delta_v, sem):
    n_rows = out_hbm.shape[0]
    rows_per_w = n_rows // NW
    n_chunks = rows_per_w // CHUNK

    wid = lax.axis_index("s") * NC + lax.axis_index("c")
    base = wid * rows_per_w

    pltpu.sync_copy(delta_hbm, delta_v)

    d0 = [delta_v[0, pl.ds(q * LANES, LANES)] for q in range(4)]
    d1 = [delta_v[1, pl.ds(q * LANES, LANES)] for q in range(4)]

    base2d = base // GSIZE

    def chunk_body(g, _):
        rbase = base + g * CHUNK
        off2d = pl.multiple_of(base2d + g * GPC, 8)
        pltpu.sync_copy(idx_hbm.at[pl.ds(off2d, GPC)], idx_v)
        copies = []
        for j in range(GPC):
            copies.append(pltpu.async_copy(
                table_hbm.at[idx_v.at[j]],
                rows_v.at[pl.ds(j * GSIZE, GSIZE)],
                sem,
            ))
        for c in copies:
            c.wait()

        def grp(i, acc):
            j = i // (GSIZE // LANES)
            k = i % (GSIZE // LANES)
            ids = idx_v[j, pl.ds(k * LANES, LANES)]
            m = (ids == TERSE_ID) | (ids == VERBOSE_ID)
            return acc | jnp.where(m, jnp.int32(1), jnp.int32(0))

        acc = lax.fori_loop(0, CHUNK // LANES,
                            grp, jnp.zeros((LANES,), jnp.int32))
        any_hit = functools.reduce(lambda a, b: a | b,
                                   [acc[r] for r in range(LANES)])

        @pl.when(any_hit > 0)
        def _fix():
            def grp_fix(i, _):
                j = i // (GSIZE // LANES)
                k = i % (GSIZE // LANES)
                ids = idx_v[j, pl.ds(k * LANES, LANES)]
                rowbase = i * LANES
                for r in range(LANES):
                    sid = ids[r]
                    row = rowbase + r

                    @pl.when(sid == TERSE_ID)
                    def _t(row=row):
                        for q in range(4):
                            cur = rows_v[row, pl.ds(q * LANES, LANES)]
                            rows_v[row, pl.ds(q * LANES, LANES)] = cur + d0[q]

                    @pl.when(sid == VERBOSE_ID)
                    def _v(row=row):
                        for q in range(4):
                            cur = rows_v[row, pl.ds(q * LANES, LANES)]
                            rows_v[row, pl.ds(q * LANES, LANES)] = cur + d1[q]

                return 0

            lax.fori_loop(0, CHUNK // LANES, grp_fix, 0)

        pltpu.sync_copy(rows_v, out_hbm.at[pl.ds(rbase, CHUNK)])
        return 0

    lax.fori_loop(0, n_chunks, chunk_body, 0)


@jax.jit
def kernel(input_ids, table, style_delta):
    b, l = input_ids.shape
    n = b * l
    idx_flat = input_ids.reshape(n)
    idx_2d = idx_flat.reshape(n // GSIZE, GSIZE)

    mesh = plsc.VectorSubcoreMesh(core_axis_name="c", subcore_axis_name="s")
    out = pl.kernel(
        _body,
        out_type=jax.ShapeDtypeStruct((n, DIM), jnp.float32),
        mesh=mesh,
        scratch_types=[
            pltpu.VMEM((GPC, GSIZE), jnp.int32),
            pltpu.VMEM((CHUNK, DIM), jnp.float32),
            pltpu.VMEM((2, DIM), jnp.float32),
            pltpu.SemaphoreType.DMA,
        ],
        compiler_params=pltpu.CompilerParams(use_tc_tiling_on_sc=False),
    )(idx_2d, table, style_delta)
    return out.reshape(b, l, DIM)


# speedup vs baseline: 1.0318x; 1.0318x over previous
import functools

import jax
import jax.numpy as jnp
from jax import lax
from jax.experimental import pallas as pl
from jax.experimental.pallas import tpu as pltpu
from jax.experimental.pallas import tpu_sc as plsc

VOCAB = 1000000
DIM = 64
TERSE_ID = 5
VERBOSE_ID = 7

NC = 2
NS = 16
NW = NC * NS
LANES = 16

GSIZE = 128
GPC = 4
CHUNK = GSIZE * GPC


def _body(idx_hbm, table_hbm, delta_hbm, out_hbm,
          idx_all, rows_v, delta_v, sem_g0, sem_g1):
    n_rows = out_hbm.shape[0]
    rows_per_w = n_rows // NW
    n_sub = rows_per_w // CHUNK
    idx_rows = rows_per_w // GSIZE

    wid = lax.axis_index("s") * NC + lax.axis_index("c")
    base = wid * rows_per_w
    base2d = pl.multiple_of(wid * idx_rows, 8)

    pltpu.sync_copy(delta_hbm, delta_v)
    pltpu.sync_copy(idx_hbm.at[pl.ds(base2d, idx_rows)], idx_all)

    d0 = [delta_v[0, pl.ds(q * LANES, LANES)] for q in range(4)]
    d1 = [delta_v[1, pl.ds(q * LANES, LANES)] for q in range(4)]

    sems = (sem_g0, sem_g1)

    def fire(s, b):
        row0 = s * GPC
        for j in range(GPC):
            pltpu.async_copy(
                table_hbm.at[idx_all.at[row0 + j]],
                rows_v.at[b].at[pl.ds(j * GSIZE, GSIZE)],
                sems[b],
            )

    def drain(b):
        for j in range(GPC):
            pltpu.make_async_copy(
                table_hbm.at[idx_all.at[j]],
                rows_v.at[b].at[pl.ds(j * GSIZE, GSIZE)],
                sems[b],
            ).wait()

    def process(s, b):
        row0 = s * GPC

        def grp(i, acc):
            j = row0 + i // (GSIZE // LANES)
            k = i % (GSIZE // LANES)
            ids = idx_all[j, pl.ds(k * LANES, LANES)]
            m = (ids == TERSE_ID) | (ids == VERBOSE_ID)
            return acc | jnp.where(m, jnp.int32(1), jnp.int32(0))

        acc = lax.fori_loop(0, CHUNK // LANES,
                            grp, jnp.zeros((LANES,), jnp.int32))
        any_hit = functools.reduce(lambda a, c: a | c,
                                   [acc[r] for r in range(LANES)])

        @pl.when(any_hit > 0)
        def _fix():
            def grp_fix(i, _):
                j = row0 + i // (GSIZE // LANES)
                k = i % (GSIZE // LANES)
                ids = idx_all[j, pl.ds(k * LANES, LANES)]
                rowbase = i * LANES
                for r in range(LANES):
                    sid = ids[r]
                    row = rowbase + r

                    @pl.when(sid == TERSE_ID)
                    def _t(row=row):
                        for q in range(4):
                            cur = rows_v[b, row, pl.ds(q * LANES, LANES)]
                            rows_v[b, row, pl.ds(q * LANES, LANES)] = (
                                cur + d0[q])

                    @pl.when(sid == VERBOSE_ID)
                    def _v(row=row):
                        for q in range(4):
                            cur = rows_v[b, row, pl.ds(q * LANES, LANES)]
                            rows_v[b, row, pl.ds(q * LANES, LANES)] = (
                                cur + d1[q])

                return 0

            lax.fori_loop(0, CHUNK // LANES, grp_fix, 0)

        pltpu.sync_copy(rows_v.at[b],
                        out_hbm.at[pl.ds(base + s * CHUNK, CHUNK)])

    fire(0, 0)

    def super_body(g, _):
        for u in range(2):
            s = 2 * g + u

            @pl.when(s + 1 < n_sub)
            def _prefetch(s=s, u=u):
                fire(s + 1, 1 - u)

            drain(u)
            process(s, u)
        return 0

    lax.fori_loop(0, n_sub // 2, super_body, 0)


@jax.jit
def kernel(input_ids, table, style_delta):
    b, l = input_ids.shape
    n = b * l
    idx_2d = input_ids.reshape(n // GSIZE, GSIZE)

    mesh = plsc.VectorSubcoreMesh(core_axis_name="c", subcore_axis_name="s")
    out = pl.kernel(
        _body,
        out_type=jax.ShapeDtypeStruct((n, DIM), jnp.float32),
        mesh=mesh,
        scratch_types=[
            pltpu.VMEM((n // NW // GSIZE, GSIZE), jnp.int32),
            pltpu.VMEM((2, CHUNK, DIM), jnp.float32),
            pltpu.VMEM((2, DIM), jnp.float32),
            pltpu.SemaphoreType.DMA,
            pltpu.SemaphoreType.DMA,
        ],
        compiler_params=pltpu.CompilerParams(use_tc_tiling_on_sc=False),
    )(idx_2d, table, style_delta)
    return out.reshape(b, l, DIM)
